# baseline (speedup 1.0000x reference)
import functools

import jax
import jax.numpy as jnp
from jax.experimental import pallas as pl
from jax.experimental.pallas import tpu as pltpu

NUM_EXPERTS = 8
HIDDEN = 1024
INTER = 1408
GROUP = 128
TOKENS = 2048
TOPK = 2

HID_W = HIDDEN // 8
INT_W = INTER // 8


def _decode_nibbles(nib):
    m = nib & 7
    e = m >> 1
    f = (m & 1).astype(jnp.float32)
    pow2 = (jnp.int32(1) << e).astype(jnp.float32) * 0.5
    mag = jnp.where(e == 0, 0.5 * f, pow2 * (1.0 + 0.5 * f))
    sign = 1.0 - 2.0 * (nib >> 3).astype(jnp.float32)
    return sign * mag


def _expand_scales(scales_compact, n_words):
    n_groups = n_words // 16
    return jnp.concatenate(
        [jnp.broadcast_to(scales_compact[:, g:g + 1],
                          (scales_compact.shape[0], 16))
         for g in range(n_groups)], axis=1)


def _moe_kernel(tki_ref, tkw_ref, x_ref,
                gp_ref, gs_ref, up_ref, us_ref, dp_ref, ds_ref,
                out_ref, wg_ref, wu_ref, wd_ref):
    e = pl.program_id(0)

    def unpack(packed_u32, scales_compact, w_scratch, n_words):
        scale_rep = _expand_scales(scales_compact, n_words)
        for k in range(8):
            nib = jax.lax.shift_right_logical(
                packed_u32, jnp.uint32(4 * k)).astype(jnp.int32) & 15
            val = _decode_nibbles(nib) * scale_rep
            w_scratch[:, k * n_words:(k + 1) * n_words] = val.astype(jnp.bfloat16)

    unpack(gp_ref[0], gs_ref[0], wg_ref, HID_W)
    unpack(up_ref[0], us_ref[0], wu_ref, HID_W)
    unpack(dp_ref[0], ds_ref[0], wd_ref, INT_W)

    w_e = jnp.sum(jnp.where(tki_ref[...] == e, tkw_ref[...], 0.0),
                  axis=1, keepdims=True)

    x = x_ref[...].astype(jnp.bfloat16)
    dn = (((1,), (1,)), ((), ()))
    g = jax.lax.dot_general(x, wg_ref[...], dn,
                            preferred_element_type=jnp.float32)
    u = jax.lax.dot_general(x, wu_ref[...], dn,
                            preferred_element_type=jnp.float32)
    h = (g * jax.nn.sigmoid(g) * u).astype(jnp.bfloat16)
    d = jax.lax.dot_general(h, wd_ref[...], dn,
                            preferred_element_type=jnp.float32)
    contrib = w_e * d

    @pl.when(e == 0)
    def _():
        out_ref[...] = contrib

    @pl.when(e > 0)
    def _():
        out_ref[...] += contrib


def _perm_inter(a):
    E = a.shape[0]
    return a.reshape(E, INT_W, 8, *a.shape[2:]).swapaxes(1, 2).reshape(
        E, INTER, *a.shape[2:])


@jax.jit
def kernel(hidden_states, top_k_index, top_k_weights,
           gate_proj_packed, gate_proj_scales,
           up_proj_packed, up_proj_scales,
           down_proj_packed, down_proj_scales):
    xr = hidden_states.reshape(TOKENS, HID_W, 8).swapaxes(1, 2).reshape(
        TOKENS, HIDDEN)

    gp = _perm_inter(gate_proj_packed)
    up = _perm_inter(up_proj_packed)
    gs = _perm_inter(gate_proj_scales.transpose(0, 2, 1))
    us = _perm_inter(up_proj_scales.transpose(0, 2, 1))
    ds = down_proj_scales.transpose(0, 2, 1)

    grid = (NUM_EXPERTS,)
    expert_block = lambda s: pl.BlockSpec((1,) + s, lambda e: (e, 0, 0))
    full = lambda s: pl.BlockSpec(s, lambda e: (0, 0))

    out = pl.pallas_call(
        _moe_kernel,
        grid=grid,
        in_specs=[
            full((TOKENS, TOPK)),
            full((TOKENS, TOPK)),
            full((TOKENS, HIDDEN)),
            expert_block((INTER, HID_W)),
            expert_block((INTER, 8)),
            expert_block((INTER, HID_W)),
            expert_block((INTER, 8)),
            expert_block((HIDDEN, INT_W)),
            expert_block((HIDDEN, 11)),
        ],
        out_specs=full((TOKENS, HIDDEN)),
        out_shape=jax.ShapeDtypeStruct((TOKENS, HIDDEN), jnp.float32),
        scratch_shapes=[
            pltpu.VMEM((INTER, HIDDEN), jnp.bfloat16),
            pltpu.VMEM((INTER, HIDDEN), jnp.bfloat16),
            pltpu.VMEM((HIDDEN, INTER), jnp.bfloat16),
        ],
    )(top_k_index, top_k_weights, xr, gp, gs, up, us, down_proj_packed, ds)
    return out


# speedup vs baseline: 1.1860x; 1.1860x over previous
import functools

import jax
import jax.numpy as jnp
from jax.experimental import pallas as pl
from jax.experimental.pallas import tpu as pltpu

NUM_EXPERTS = 8
HIDDEN = 1024
INTER = 1408
GROUP = 128
TOKENS = 2048
TOPK = 2

HID_W = HIDDEN // 8
INT_W = INTER // 8


def _decode_nibbles(nib):
    m = nib & 7
    e = m >> 1
    f = (m & 1).astype(jnp.float32)
    pow2 = (jnp.int32(1) << e).astype(jnp.float32) * 0.5
    mag = jnp.where(e == 0, 0.5 * f, pow2 * (1.0 + 0.5 * f))
    sign = 1.0 - 2.0 * (nib >> 3).astype(jnp.float32)
    return sign * mag


def _moe_kernel(tki_ref, tkw_ref, x_ref,
                gp_ref, gs_ref, up_ref, us_ref, dp_ref, ds_ref,
                out_ref, wg_ref, wu_ref, wd_ref):
    e = pl.program_id(0)

    def unpack(packed_u32, scale_rep_ref, w_scratch, n_words):
        scale_rep = scale_rep_ref.astype(jnp.float32)
        for k in range(8):
            nib = jax.lax.shift_right_logical(
                packed_u32, jnp.uint32(4 * k)).astype(jnp.int32) & 15
            val = _decode_nibbles(nib) * scale_rep
            w_scratch[:, k * n_words:(k + 1) * n_words] = val.astype(jnp.bfloat16)

    unpack(gp_ref[0], gs_ref[0], wg_ref, HID_W)
    unpack(up_ref[0], us_ref[0], wu_ref, HID_W)
    unpack(dp_ref[0], ds_ref[0], wd_ref, INT_W)

    w_e = jnp.sum(jnp.where(tki_ref[...] == e, tkw_ref[...], 0.0),
                  axis=1, keepdims=True)

    x = x_ref[...].astype(jnp.bfloat16)
    dn = (((1,), (1,)), ((), ()))
    g = jax.lax.dot_general(x, wg_ref[...], dn,
                            preferred_element_type=jnp.float32)
    u = jax.lax.dot_general(x, wu_ref[...], dn,
                            preferred_element_type=jnp.float32)
    h = (g * jax.nn.sigmoid(g) * u).astype(jnp.bfloat16)
    d = jax.lax.dot_general(h, wd_ref[...], dn,
                            preferred_element_type=jnp.float32)
    contrib = w_e * d

    @pl.when(e == 0)
    def _():
        out_ref[...] = contrib

    @pl.when(e > 0)
    def _():
        out_ref[...] += contrib


def _perm_inter(a):
    E = a.shape[0]
    return a.reshape(E, INT_W, 8, *a.shape[2:]).swapaxes(1, 2).reshape(
        E, INTER, *a.shape[2:])


@jax.jit
def kernel(hidden_states, top_k_index, top_k_weights,
           gate_proj_packed, gate_proj_scales,
           up_proj_packed, up_proj_scales,
           down_proj_packed, down_proj_scales):
    xr = hidden_states.reshape(TOKENS, HID_W, 8).swapaxes(1, 2).reshape(
        TOKENS, HIDDEN)

    gp = _perm_inter(gate_proj_packed)
    up = _perm_inter(up_proj_packed)
    gs = jnp.repeat(_perm_inter(gate_proj_scales.transpose(0, 2, 1)), 16,
                    axis=2).astype(jnp.bfloat16)
    us = jnp.repeat(_perm_inter(up_proj_scales.transpose(0, 2, 1)), 16,
                    axis=2).astype(jnp.bfloat16)
    ds = jnp.repeat(down_proj_scales.transpose(0, 2, 1), 16,
                    axis=2).astype(jnp.bfloat16)

    grid = (NUM_EXPERTS,)
    expert_block = lambda s: pl.BlockSpec((1,) + s, lambda e: (e, 0, 0))
    full = lambda s: pl.BlockSpec(s, lambda e: (0, 0))

    out = pl.pallas_call(
        _moe_kernel,
        grid=grid,
        in_specs=[
            full((TOKENS, TOPK)),
            full((TOKENS, TOPK)),
            full((TOKENS, HIDDEN)),
            expert_block((INTER, HID_W)),
            expert_block((INTER, HID_W)),
            expert_block((INTER, HID_W)),
            expert_block((INTER, HID_W)),
            expert_block((HIDDEN, INT_W)),
            expert_block((HIDDEN, INT_W)),
        ],
        out_specs=full((TOKENS, HIDDEN)),
        out_shape=jax.ShapeDtypeStruct((TOKENS, HIDDEN), jnp.float32),
        scratch_shapes=[
            pltpu.VMEM((INTER, HIDDEN), jnp.bfloat16),
            pltpu.VMEM((INTER, HIDDEN), jnp.bfloat16),
            pltpu.VMEM((HIDDEN, INTER), jnp.bfloat16),
        ],
    )(top_k_index, top_k_weights, xr, gp, gs, up, us, down_proj_packed, ds)
    return out
